# Initial kernel scaffold; baseline (speedup 1.0000x reference)
import jax
import jax.numpy as jnp
from jax.experimental import pallas as pl


def kernel(x, edge_index, edge_type, W1, root1, bias1, W2, root2, bias2):
    raise NotImplementedError("write your pallas kernel here")


# speedup vs baseline: 2.3200x; 2.3200x over previous
import functools

import jax
import jax.numpy as jnp
from jax import lax
from jax.experimental import pallas as pl
from jax.experimental.pallas import tpu as pltpu
from jax.experimental.pallas import tpu_sc as plsc

N = 10000
E = 320000
R = 16
F = 128

NC = 2
NS = 16
NW = NC * NS

EPW = 10240
EP = EPW * NW
ERW = EPW // 128
ERT = EP // 128
ERC = ERT // NS

NR = N * R
NRP = 161792
NRS = NRP // NS
NP = 10112
NPS = NP // NS


def _mesh():
    return plsc.VectorSubcoreMesh(core_axis_name="c", subcore_axis_name="s",
                                  num_cores=NC, num_subcores=NS)


@functools.partial(
    pl.kernel,
    out_type=jax.ShapeDtypeStruct((NC, NP, F), jnp.float32),
    mesh=_mesh(),
    scratch_types=[
        pltpu.VMEM((1, 128), jnp.int32),
        pltpu.VMEM((1, 128), jnp.int32),
        pltpu.VMEM((1, 128), jnp.int32),
        pltpu.VMEM((128, F), jnp.float32),
        pltpu.VMEM((128,), jnp.float32),
        pltpu.VMEM((128,), jnp.float32),
        pltpu.VMEM((128,), jnp.float32),
        pltpu.VMEM_SHARED((NP, F), jnp.float32),
        pltpu.VMEM_SHARED((NRP,), jnp.float32),
        pltpu.SemaphoreType.DMA,
    ],
)
def _sc_layer(table_hbm, gidx_hbm, comb_hbm, dst_hbm, zn_hbm, znr_hbm,
              ones_hbm, agg_out, gidx_v, comb_v, dst_v, rows_v, cnt_v, inv_v,
              ones_v, agg_sh, cnt_sh, sem):
    cid = lax.axis_index("c")
    sid = lax.axis_index("s")
    wid = sid * NC + cid

    pltpu.sync_copy(zn_hbm, agg_sh.at[pl.ds(sid * NPS, NPS)])
    pltpu.sync_copy(znr_hbm, cnt_sh.at[pl.ds(sid * NRS, NRS)])
    pltpu.sync_copy(ones_hbm, ones_v)
    plsc.subcore_barrier()

    def hist_body(c, _):
        rb = sid * ERC + c
        pltpu.sync_copy(comb_hbm.at[pl.ds(rb, 1)], comb_v)
        pltpu.sync_copy(ones_v, cnt_sh.at[comb_v.at[0]], add=True)
        return 0

    lax.fori_loop(0, ERC, hist_body, 0, unroll=False)
    plsc.subcore_barrier()

    def main_body(c, _):
        rb = wid * ERW + c
        pltpu.sync_copy(gidx_hbm.at[pl.ds(rb, 1)], gidx_v)
        pltpu.sync_copy(comb_hbm.at[pl.ds(rb, 1)], comb_v)
        pltpu.sync_copy(dst_hbm.at[pl.ds(rb, 1)], dst_v)
        pltpu.async_copy(table_hbm.at[gidx_v.at[0]], rows_v, sem)
        pltpu.sync_copy(cnt_sh.at[comb_v.at[0]], cnt_v)
        for t in range(8):
            c16 = cnt_v[pl.ds(t * 16, 16)]
            inv_v[pl.ds(t * 16, 16)] = 1.0 / jnp.maximum(c16, 1.0)
        pltpu.make_async_copy(table_hbm.at[gidx_v.at[0]], rows_v, sem).wait()
        for t in range(8):
            iv16 = inv_v[pl.ds(t * 16, 16)]
            for u in range(16):
                e = t * 16 + u
                bvec = jnp.full((16,), iv16[u], jnp.float32)
                row = rows_v.at[e]
                for q in range(8):
                    row[pl.ds(q * 16, 16)] = row[pl.ds(q * 16, 16)] * bvec
        pltpu.sync_copy(rows_v, agg_sh.at[dst_v.at[0]], add=True)
        return 0

    lax.fori_loop(0, ERW, main_body, 0, unroll=False)
    plsc.subcore_barrier()
    pltpu.sync_copy(agg_sh.at[pl.ds(sid * NPS, NPS)],
                    agg_out.at[cid, pl.ds(sid * NPS, NPS)])


@functools.partial(
    pl.kernel,
    out_type=(jax.ShapeDtypeStruct((EP, F), jnp.float32),
              jax.ShapeDtypeStruct((EP, F), jnp.float32)),
    mesh=_mesh(),
    scratch_types=[
        pltpu.VMEM((1, 128), jnp.int32),
        pltpu.VMEM((1, 128), jnp.int32),
        pltpu.VMEM((128, F), jnp.float32),
        pltpu.VMEM((128, F), jnp.float32),
        pltpu.SemaphoreType.DMA,
    ],
)
def _sc_gather2(table_hbm, aidx_hbm, bidx_hbm, a_out, b_out,
                aidx_v, bidx_v, arows_v, brows_v, sem):
    cid = lax.axis_index("c")
    sid = lax.axis_index("s")
    wid = sid * NC + cid

    def body(c, _):
        rb = wid * ERW + c
        base = rb * 128
        pltpu.sync_copy(aidx_hbm.at[pl.ds(rb, 1)], aidx_v)
        pltpu.sync_copy(bidx_hbm.at[pl.ds(rb, 1)], bidx_v)
        pltpu.async_copy(table_hbm.at[aidx_v.at[0]], arows_v, sem)
        pltpu.async_copy(table_hbm.at[bidx_v.at[0]], brows_v, sem)
        pltpu.make_async_copy(table_hbm.at[aidx_v.at[0]], arows_v, sem).wait()
        pltpu.make_async_copy(table_hbm.at[bidx_v.at[0]], brows_v, sem).wait()
        pltpu.sync_copy(arows_v, a_out.at[pl.ds(base, 128)])
        pltpu.sync_copy(brows_v, b_out.at[pl.ds(base, 128)])
        return 0

    lax.fori_loop(0, ERW, body, 0, unroll=False)


BN = 400


def _dense_body(x_ref, w_ref, o_ref):
    o_ref[0] = jnp.dot(x_ref[...], w_ref[0], preferred_element_type=jnp.float32)


def _dense_rel(x, W):
    return pl.pallas_call(
        _dense_body,
        grid=(R, N // BN),
        in_specs=[
            pl.BlockSpec((BN, F), lambda r, n: (n, 0)),
            pl.BlockSpec((1, F, F), lambda r, n: (r, 0, 0)),
        ],
        out_specs=pl.BlockSpec((1, BN, F), lambda r, n: (r, n, 0)),
        out_shape=jax.ShapeDtypeStruct((R, N, F), jnp.float32),
    )(x, W)


def _combine_body(relu, agg_ref, x_ref, root_ref, bias_ref, o_ref):
    out = (agg_ref[0] + agg_ref[1]
           + jnp.dot(x_ref[...], root_ref[...],
                     preferred_element_type=jnp.float32)
           + bias_ref[...])
    if relu:
        out = jnp.maximum(out, 0.0)
    o_ref[...] = out


def _combine(aggP, x, root, bias, relu):
    return pl.pallas_call(
        functools.partial(_combine_body, relu),
        grid=(N // BN,),
        in_specs=[
            pl.BlockSpec((NC, BN, F), lambda n: (0, n, 0)),
            pl.BlockSpec((BN, F), lambda n: (n, 0)),
            pl.BlockSpec((F, F), lambda n: (0, 0)),
            pl.BlockSpec((1, F), lambda n: (0, 0)),
        ],
        out_specs=pl.BlockSpec((BN, F), lambda n: (n, 0)),
        out_shape=jax.ShapeDtypeStruct((N, F), jnp.float32),
    )(aggP, x, root, bias)


BE = 2048


def _rowdot_body(a_ref, b_ref, o_ref):
    o_ref[...] = jnp.sum(a_ref[...] * b_ref[...], axis=1, keepdims=True)


def _rowdot(a, b):
    return pl.pallas_call(
        _rowdot_body,
        grid=(EP // BE,),
        in_specs=[
            pl.BlockSpec((BE, F), lambda i: (i, 0)),
            pl.BlockSpec((BE, F), lambda i: (i, 0)),
        ],
        out_specs=pl.BlockSpec((BE, 1), lambda i: (i, 0)),
        out_shape=jax.ShapeDtypeStruct((EP, 1), jnp.float32),
    )(a, b)


def _layer(x, W, root, bias, gidx, comb, dstp, zn, znr, ones1, relu):
    h_all = _dense_rel(x, W).reshape(R * N, F)
    aggP = _sc_layer(h_all, gidx, comb, dstp, zn, znr, ones1)
    return _combine(aggP, x, root, bias, relu)


def kernel(x, edge_index, edge_type, W1, root1, bias1, W2, root2, bias2):
    src = edge_index[0]
    dst = edge_index[1]
    pad = EP - E
    i2d = lambda a: a.reshape(ERT, 128)
    gidx = i2d(jnp.pad(edge_type * N + src, (0, pad)))
    comb = i2d(jnp.pad(dst * R + edge_type, (0, pad), constant_values=NR))
    dstp = i2d(jnp.pad(dst, (0, pad), constant_values=N))
    srcp = i2d(jnp.pad(src, (0, pad)))
    dstg = i2d(jnp.pad(dst, (0, pad)))

    zn = jnp.zeros((NPS, F), jnp.float32)
    znr = jnp.zeros((NRS,), jnp.float32)
    ones1 = jnp.ones((128,), jnp.float32)

    bias1r = bias1.reshape(1, F)
    bias2r = bias2.reshape(1, F)

    h = _layer(x, W1, root1, bias1r, gidx, comb, dstp, zn, znr, ones1, True)
    z = _layer(h, W2, root2, bias2r, gidx, comb, dstp, zn, znr, ones1, False)

    src_z, dst_z = _sc_gather2(z, srcp, dstg)
    scores = _rowdot(src_z, dst_z)
    return scores[:E, 0]


# speedup vs baseline: 3.0025x; 1.2942x over previous
import functools

import jax
import jax.numpy as jnp
from jax import lax
from jax.experimental import pallas as pl
from jax.experimental.pallas import tpu as pltpu
from jax.experimental.pallas import tpu_sc as plsc

N = 10000
E = 320000
R = 16
F = 128

NC = 2
NS = 16
NW = NC * NS

EPW = 10240
EP = EPW * NW
ERW = EPW // 128
ERT = EP // 128
ERC = ERT // NS

NR = N * R
NRP = 161792
NRS = NRP // NS
NP = 10112
NPS = NP // NS

HB = 16
IB = ERC // HB
GB = ERW // HB


def _mesh():
    return plsc.VectorSubcoreMesh(core_axis_name="c", subcore_axis_name="s",
                                  num_cores=NC, num_subcores=NS)


@functools.partial(
    pl.kernel,
    out_type=jax.ShapeDtypeStruct((EP,), jnp.float32),
    mesh=_mesh(),
    scratch_types=[
        pltpu.VMEM((HB, 128), jnp.int32),
        pltpu.VMEM((128,), jnp.float32),
        pltpu.VMEM((HB * 128,), jnp.float32),
        pltpu.VMEM((HB * 128,), jnp.float32),
        pltpu.VMEM_SHARED((NRP,), jnp.float32),
        pltpu.SemaphoreType.DMA,
        pltpu.SemaphoreType.DMA,
    ],
)
def _sc_hist_inv(comb_hbm, znr_hbm, ones_hbm, inv_out, cblk_v, ones_v,
                 cntb_v, invb_v, cnt_sh, semh, semg):
    cid = lax.axis_index("c")
    sid = lax.axis_index("s")
    wid = sid * NC + cid

    pltpu.sync_copy(znr_hbm, cnt_sh.at[pl.ds(sid * NRS, NRS)])
    pltpu.sync_copy(ones_hbm, ones_v)
    plsc.subcore_barrier()

    def hist_body(b, _):
        pltpu.sync_copy(comb_hbm.at[pl.ds(sid * ERC + b * HB, HB)], cblk_v)
        descs = [pltpu.async_copy(ones_v, cnt_sh.at[cblk_v.at[j]], semh,
                                  add=True) for j in range(HB)]
        for d in descs:
            d.wait()
        return 0

    lax.fori_loop(0, IB, hist_body, 0, unroll=False)
    plsc.subcore_barrier()

    def inv_body(b, _):
        pltpu.sync_copy(comb_hbm.at[pl.ds(wid * ERW + b * HB, HB)], cblk_v)
        descs = [pltpu.async_copy(cnt_sh.at[cblk_v.at[j]],
                                  cntb_v.at[pl.ds(j * 128, 128)], semg)
                 for j in range(HB)]
        for d in descs:
            d.wait()
        for t in range(HB * 8):
            c16 = cntb_v[pl.ds(t * 16, 16)]
            invb_v[pl.ds(t * 16, 16)] = 1.0 / jnp.maximum(c16, 1.0)
        pltpu.sync_copy(
            invb_v, inv_out.at[pl.ds(wid * EPW + b * HB * 128, HB * 128)])
        return 0

    lax.fori_loop(0, GB, inv_body, 0, unroll=False)


def _scale_rows(buf, invp, base):
    def tbody(t, _):
        iv16 = invp[pl.ds(base + t * 16, 16)]
        for u in range(16):
            row = buf.at[t * 16 + u]
            bvec = jnp.full((16,), iv16[u], jnp.float32)
            for q in range(8):
                row[pl.ds(q * 16, 16)] = row[pl.ds(q * 16, 16)] * bvec
        return 0

    lax.fori_loop(0, 8, tbody, 0, unroll=False)


@functools.partial(
    pl.kernel,
    out_type=jax.ShapeDtypeStruct((NC, NP, F), jnp.float32),
    mesh=_mesh(),
    scratch_types=[
        pltpu.VMEM((8, 2, 128), jnp.int32),
        pltpu.VMEM((1024,), jnp.float32),
        pltpu.VMEM((128, F), jnp.float32),
        pltpu.VMEM((128, F), jnp.float32),
        pltpu.VMEM_SHARED((NP, F), jnp.float32),
        pltpu.SemaphoreType.DMA,
        pltpu.SemaphoreType.DMA,
    ],
)
def _sc_layer(table_hbm, idx2_hbm, inv_hbm, zn_hbm, agg_out,
              ix_v, invp_v, bufa_v, bufb_v, agg_sh, semg, sems):
    cid = lax.axis_index("c")
    sid = lax.axis_index("s")
    wid = sid * NC + cid

    pltpu.sync_copy(zn_hbm, agg_sh.at[pl.ds(sid * NPS, NPS)])
    plsc.subcore_barrier()

    def body(k, _):
        rb = wid * ERW + 8 * k
        pltpu.sync_copy(idx2_hbm.at[pl.ds(rb, 8)], ix_v)
        pltpu.sync_copy(inv_hbm.at[pl.ds(wid * EPW + k * 1024, 1024)],
                        invp_v)
        bufs = [bufa_v, bufb_v]
        g = {0: pltpu.async_copy(table_hbm.at[ix_v.at[0, 0]], bufa_v, semg)}
        s = {}
        for r in range(8):
            buf = bufs[r % 2]
            g[r].wait()
            _scale_rows(buf, invp_v, r * 128)
            s[r] = pltpu.async_copy(buf, agg_sh.at[ix_v.at[r, 1]], sems,
                                    add=True)
            if r + 1 < 8:
                if r >= 1:
                    s[r - 1].wait()
                g[r + 1] = pltpu.async_copy(table_hbm.at[ix_v.at[r + 1, 0]],
                                            bufs[(r + 1) % 2], semg)
        s[6].wait()
        s[7].wait()
        return 0

    lax.fori_loop(0, ERW // 8, body, 0, unroll=False)
    plsc.subcore_barrier()
    pltpu.sync_copy(agg_sh.at[pl.ds(sid * NPS, NPS)],
                    agg_out.at[cid, pl.ds(sid * NPS, NPS)])


@functools.partial(
    pl.kernel,
    out_type=(jax.ShapeDtypeStruct((EP, F), jnp.float32),
              jax.ShapeDtypeStruct((EP, F), jnp.float32)),
    mesh=_mesh(),
    scratch_types=[
        pltpu.VMEM((8, 2, 128), jnp.int32),
        pltpu.VMEM((128, F), jnp.float32),
        pltpu.VMEM((128, F), jnp.float32),
        pltpu.VMEM((128, F), jnp.float32),
        pltpu.VMEM((128, F), jnp.float32),
        pltpu.SemaphoreType.DMA,
        pltpu.SemaphoreType.DMA,
    ],
)
def _sc_gather2(table_hbm, didx2_hbm, a_out, b_out,
                ix_v, a0_v, b0_v, a1_v, b1_v, semg, semw):
    cid = lax.axis_index("c")
    sid = lax.axis_index("s")
    wid = sid * NC + cid

    def body(k, _):
        rb = wid * ERW + 8 * k
        pltpu.sync_copy(didx2_hbm.at[pl.ds(rb, 8)], ix_v)
        apair = [a0_v, a1_v]
        bpair = [b0_v, b1_v]
        g = {0: [pltpu.async_copy(table_hbm.at[ix_v.at[0, 0]], a0_v, semg),
                 pltpu.async_copy(table_hbm.at[ix_v.at[0, 1]], b0_v, semg)]}
        w = {}
        for r in range(8):
            av = apair[r % 2]
            bv = bpair[r % 2]
            for d in g[r]:
                d.wait()
            w[r] = [
                pltpu.async_copy(av, a_out.at[pl.ds((rb + r) * 128, 128)],
                                 semw),
                pltpu.async_copy(bv, b_out.at[pl.ds((rb + r) * 128, 128)],
                                 semw)]
            if r + 1 < 8:
                if r >= 1:
                    for d in w[r - 1]:
                        d.wait()
                nav = apair[(r + 1) % 2]
                nbv = bpair[(r + 1) % 2]
                g[r + 1] = [
                    pltpu.async_copy(table_hbm.at[ix_v.at[r + 1, 0]], nav,
                                     semg),
                    pltpu.async_copy(table_hbm.at[ix_v.at[r + 1, 1]], nbv,
                                     semg)]
        for d in w[6] + w[7]:
            d.wait()
        return 0

    lax.fori_loop(0, ERW // 8, body, 0, unroll=False)


BN = 400


def _dense_body(x_ref, w_ref, o_ref):
    o_ref[0] = jnp.dot(x_ref[...], w_ref[0], preferred_element_type=jnp.float32)


def _dense_rel(x, W):
    return pl.pallas_call(
        _dense_body,
        grid=(R, N // BN),
        in_specs=[
            pl.BlockSpec((BN, F), lambda r, n: (n, 0)),
            pl.BlockSpec((1, F, F), lambda r, n: (r, 0, 0)),
        ],
        out_specs=pl.BlockSpec((1, BN, F), lambda r, n: (r, n, 0)),
        out_shape=jax.ShapeDtypeStruct((R, N, F), jnp.float32),
    )(x, W)


def _combine_body(relu, agg_ref, x_ref, root_ref, bias_ref, o_ref):
    out = (agg_ref[0] + agg_ref[1]
           + jnp.dot(x_ref[...], root_ref[...],
                     preferred_element_type=jnp.float32)
           + bias_ref[...])
    if relu:
        out = jnp.maximum(out, 0.0)
    o_ref[...] = out


def _combine(aggP, x, root, bias, relu):
    return pl.pallas_call(
        functools.partial(_combine_body, relu),
        grid=(N // BN,),
        in_specs=[
            pl.BlockSpec((NC, BN, F), lambda n: (0, n, 0)),
            pl.BlockSpec((BN, F), lambda n: (n, 0)),
            pl.BlockSpec((F, F), lambda n: (0, 0)),
            pl.BlockSpec((1, F), lambda n: (0, 0)),
        ],
        out_specs=pl.BlockSpec((BN, F), lambda n: (n, 0)),
        out_shape=jax.ShapeDtypeStruct((N, F), jnp.float32),
    )(aggP, x, root, bias)


BE = 2048


def _rowdot_body(a_ref, b_ref, o_ref):
    o_ref[...] = jnp.sum(a_ref[...] * b_ref[...], axis=1, keepdims=True)


def _rowdot(a, b):
    return pl.pallas_call(
        _rowdot_body,
        grid=(EP // BE,),
        in_specs=[
            pl.BlockSpec((BE, F), lambda i: (i, 0)),
            pl.BlockSpec((BE, F), lambda i: (i, 0)),
        ],
        out_specs=pl.BlockSpec((BE, 1), lambda i: (i, 0)),
        out_shape=jax.ShapeDtypeStruct((EP, 1), jnp.float32),
    )(a, b)


def _layer(x, W, root, bias, idx2, inv_e, zn, relu):
    h_all = _dense_rel(x, W).reshape(R * N, F)
    aggP = _sc_layer(h_all, idx2, inv_e, zn)
    return _combine(aggP, x, root, bias, relu)


def kernel(x, edge_index, edge_type, W1, root1, bias1, W2, root2, bias2):
    src = edge_index[0]
    dst = edge_index[1]
    pad = EP - E
    i2d = lambda a: a.reshape(ERT, 128)
    gidx = i2d(jnp.pad(edge_type * N + src, (0, pad)))
    comb = i2d(jnp.pad(dst * R + edge_type, (0, pad), constant_values=NR))
    dstp = i2d(jnp.pad(dst, (0, pad), constant_values=N))
    srcp = i2d(jnp.pad(src, (0, pad)))
    dstg = i2d(jnp.pad(dst, (0, pad)))
    idx2 = jnp.stack([gidx, dstp], axis=1)
    didx2 = jnp.stack([srcp, dstg], axis=1)

    zn = jnp.zeros((NPS, F), jnp.float32)
    znr = jnp.zeros((NRS,), jnp.float32)
    ones1 = jnp.ones((128,), jnp.float32)

    inv_e = _sc_hist_inv(comb, znr, ones1)

    bias1r = bias1.reshape(1, F)
    bias2r = bias2.reshape(1, F)

    h = _layer(x, W1, root1, bias1r, idx2, inv_e, zn, True)
    z = _layer(h, W2, root2, bias2r, idx2, inv_e, zn, False)

    src_z, dst_z = _sc_gather2(z, didx2)
    scores = _rowdot(src_z, dst_z)
    return scores[:E, 0]
